# baseline (speedup 1.0000x reference)
import functools

import jax
import jax.numpy as jnp
from jax import lax
from jax.experimental import pallas as pl
from jax.experimental.pallas import tpu as pltpu
from jax.experimental.pallas import tpu_sc as plsc

RATIO = 16
LANES = 16
NUM_WORKERS = 32
REP = 8


def _build_sc_kernel(batch, seq_len, hidden):
    num_global = seq_len // RATIO
    rows = batch * num_global
    rpw = rows // NUM_WORKERS
    wpw = rpw
    n_out = rpw // REP
    mesh = plsc.VectorSubcoreMesh(core_axis_name="c", subcore_axis_name="s")

    @functools.partial(
        pl.kernel,
        mesh=mesh,
        out_type=[
            jax.ShapeDtypeStruct((rows * hidden,), jnp.float32),
            jax.ShapeDtypeStruct((rows,), jnp.int32),
        ],
        scratch_types=[
            pltpu.VMEM(((1 + REP) * hidden,), jnp.float32),
            pltpu.VMEM((RATIO * wpw,), jnp.int32),
            pltpu.VMEM((wpw,), jnp.int32),
            pltpu.SemaphoreType.DMA,
            pltpu.SemaphoreType.DMA,
            pltpu.SemaphoreType.DMA,
            pltpu.SemaphoreType.DMA,
        ],
    )
    def sc_kernel(mask_hbm, embeds_hbm, out_hbm, gmask_hbm,
                  slab_v, mask_v, gout_v,
                  sem_r0, sem_r1, sem_mask, sem_out):
        wid = lax.axis_index("s") * 2 + lax.axis_index("c")
        base = wid * rpw

        cp_r1 = pltpu.async_copy(
            embeds_hbm.at[1], slab_v.at[pl.ds(0, hidden)], sem_r1)
        cp_r0 = pltpu.async_copy(
            embeds_hbm.at[0], slab_v.at[pl.ds(hidden, hidden)], sem_r0)
        cp_mask = pltpu.async_copy(
            mask_hbm.at[pl.ds(wid * RATIO * wpw, RATIO * wpw)], mask_v,
            sem_mask)

        cp_r0.wait()
        for v in range(hidden // LANES):
            x = slab_v[pl.ds(hidden + v * LANES, LANES)]
            for r in range(2, REP + 1):
                slab_v[pl.ds(r * hidden + v * LANES, LANES)] = x

        cp_r1.wait()
        first_src = jnp.where(base % num_global == 0, 0, hidden)
        out_copies = [
            pltpu.async_copy(
                slab_v.at[pl.ds(first_src, REP * hidden)],
                out_hbm.at[pl.ds(base * hidden, REP * hidden)],
                sem_out)
        ]
        for i in range(1, n_out):
            out_copies.append(pltpu.async_copy(
                slab_v.at[pl.ds(hidden, REP * hidden)],
                out_hbm.at[pl.ds((base + i * REP) * hidden, REP * hidden)],
                sem_out))

        cp_mask.wait()
        for g in range(wpw // LANES):
            acc = mask_v[pl.ds(g * LANES, LANES)]
            for k in range(1, RATIO):
                acc = jnp.minimum(
                    acc, mask_v[pl.ds(k * wpw + g * LANES, LANES)])
            gout_v[pl.ds(g * LANES, LANES)] = acc
        pltpu.sync_copy(gout_v, gmask_hbm.at[pl.ds(base, wpw)])

        for cp in out_copies:
            cp.wait()

    return sc_kernel


def kernel(token_ids, padding_mask, embeds):
    batch, seq_len = token_ids.shape
    hidden = embeds.shape[1]
    num_global = seq_len // RATIO
    rows = batch * seq_len // RATIO
    wpw = rows // NUM_WORKERS
    mask_t = (
        padding_mask.astype(jnp.int32)
        .reshape(NUM_WORKERS, wpw, RATIO)
        .transpose(0, 2, 1)
        .reshape(batch * seq_len)
    )
    out_flat, gmask = _build_sc_kernel(batch, seq_len, hidden)(mask_t, embeds)
    out = out_flat.reshape(batch, num_global, hidden)
    gmask = gmask.reshape(batch, num_global).astype(jnp.bool_)
    return out, gmask


# speedup vs baseline: 1.0030x; 1.0030x over previous
import functools

import jax
import jax.numpy as jnp
from jax import lax
from jax.experimental import pallas as pl
from jax.experimental.pallas import tpu as pltpu
from jax.experimental.pallas import tpu_sc as plsc

RATIO = 16
LANES = 16
NUM_WORKERS = 32
REP = 1


def _build_sc_kernel(batch, seq_len, hidden):
    num_global = seq_len // RATIO
    rows = batch * num_global
    rpw = rows // NUM_WORKERS
    wpw = rpw
    n_out = rpw // REP
    mesh = plsc.VectorSubcoreMesh(core_axis_name="c", subcore_axis_name="s")

    @functools.partial(
        pl.kernel,
        mesh=mesh,
        out_type=[
            jax.ShapeDtypeStruct((rows * hidden,), jnp.float32),
            jax.ShapeDtypeStruct((rows,), jnp.int32),
        ],
        scratch_types=[
            pltpu.VMEM(((1 + REP) * hidden,), jnp.float32),
            pltpu.VMEM((RATIO * wpw,), jnp.int32),
            pltpu.VMEM((wpw,), jnp.int32),
            pltpu.SemaphoreType.DMA,
            pltpu.SemaphoreType.DMA,
            pltpu.SemaphoreType.DMA,
            pltpu.SemaphoreType.DMA,
        ],
    )
    def sc_kernel(mask_hbm, embeds_hbm, out_hbm, gmask_hbm,
                  slab_v, mask_v, gout_v,
                  sem_r0, sem_r1, sem_mask, sem_out):
        wid = lax.axis_index("s") * 2 + lax.axis_index("c")
        base = wid * rpw

        cp_r1 = pltpu.async_copy(
            embeds_hbm.at[1], slab_v.at[pl.ds(0, hidden)], sem_r1)
        cp_r0 = pltpu.async_copy(
            embeds_hbm.at[0], slab_v.at[pl.ds(hidden, hidden)], sem_r0)
        cp_mask = pltpu.async_copy(
            mask_hbm.at[pl.ds(wid * RATIO * wpw, RATIO * wpw)], mask_v,
            sem_mask)

        cp_r0.wait()
        for v in range(hidden // LANES):
            x = slab_v[pl.ds(hidden + v * LANES, LANES)]
            for r in range(2, REP + 1):
                slab_v[pl.ds(r * hidden + v * LANES, LANES)] = x

        cp_r1.wait()
        first_src = jnp.where(base % num_global == 0, 0, hidden)
        out_copies = [
            pltpu.async_copy(
                slab_v.at[pl.ds(first_src, REP * hidden)],
                out_hbm.at[pl.ds(base * hidden, REP * hidden)],
                sem_out)
        ]
        for i in range(1, n_out):
            out_copies.append(pltpu.async_copy(
                slab_v.at[pl.ds(hidden, REP * hidden)],
                out_hbm.at[pl.ds((base + i * REP) * hidden, REP * hidden)],
                sem_out))

        cp_mask.wait()
        for g in range(wpw // LANES):
            acc = mask_v[pl.ds(g * LANES, LANES)]
            for k in range(1, RATIO):
                acc = jnp.minimum(
                    acc, mask_v[pl.ds(k * wpw + g * LANES, LANES)])
            gout_v[pl.ds(g * LANES, LANES)] = acc
        pltpu.sync_copy(gout_v, gmask_hbm.at[pl.ds(base, wpw)])

        for cp in out_copies:
            cp.wait()

    return sc_kernel


def kernel(token_ids, padding_mask, embeds):
    batch, seq_len = token_ids.shape
    hidden = embeds.shape[1]
    num_global = seq_len // RATIO
    rows = batch * seq_len // RATIO
    wpw = rows // NUM_WORKERS
    mask_t = (
        padding_mask.astype(jnp.int32)
        .reshape(NUM_WORKERS, wpw, RATIO)
        .transpose(0, 2, 1)
        .reshape(batch * seq_len)
    )
    out_flat, gmask = _build_sc_kernel(batch, seq_len, hidden)(mask_t, embeds)
    out = out_flat.reshape(batch, num_global, hidden)
    gmask = gmask.reshape(batch, num_global).astype(jnp.bool_)
    return out, gmask


# speedup vs baseline: 1.0190x; 1.0160x over previous
import functools

import jax
import jax.numpy as jnp
from jax import lax
from jax.experimental import pallas as pl
from jax.experimental.pallas import tpu as pltpu
from jax.experimental.pallas import tpu_sc as plsc

RATIO = 16
LANES = 16
NUM_WORKERS = 32
REP = 4


def _build_sc_kernel(batch, seq_len, hidden):
    num_global = seq_len // RATIO
    rows = batch * num_global
    rpw = rows // NUM_WORKERS
    wpw = rpw
    n_out = rpw // REP
    mesh = plsc.VectorSubcoreMesh(core_axis_name="c", subcore_axis_name="s")

    @functools.partial(
        pl.kernel,
        mesh=mesh,
        out_type=[
            jax.ShapeDtypeStruct((rows * hidden,), jnp.float32),
            jax.ShapeDtypeStruct((rows,), jnp.int32),
        ],
        scratch_types=[
            pltpu.VMEM(((1 + REP) * hidden,), jnp.float32),
            pltpu.VMEM((RATIO * wpw,), jnp.int32),
            pltpu.VMEM((wpw,), jnp.int32),
            pltpu.SemaphoreType.DMA,
            pltpu.SemaphoreType.DMA,
            pltpu.SemaphoreType.DMA,
            pltpu.SemaphoreType.DMA,
        ],
    )
    def sc_kernel(mask_hbm, embeds_hbm, out_hbm, gmask_hbm,
                  slab_v, mask_v, gout_v,
                  sem_r0, sem_r1, sem_mask, sem_out):
        wid = lax.axis_index("s") * 2 + lax.axis_index("c")
        base = wid * rpw

        cp_r1 = pltpu.async_copy(
            embeds_hbm.at[1], slab_v.at[pl.ds(0, hidden)], sem_r1)
        cp_r0 = pltpu.async_copy(
            embeds_hbm.at[0], slab_v.at[pl.ds(hidden, hidden)], sem_r0)
        cp_mask = pltpu.async_copy(
            mask_hbm.at[pl.ds(wid * RATIO * wpw, RATIO * wpw)], mask_v,
            sem_mask)

        cp_r0.wait()
        for v in range(hidden // LANES):
            x = slab_v[pl.ds(hidden + v * LANES, LANES)]
            for r in range(2, REP + 1):
                slab_v[pl.ds(r * hidden + v * LANES, LANES)] = x

        cp_r1.wait()
        first_src = jnp.where(base % num_global == 0, 0, hidden)
        out_copies = [
            pltpu.async_copy(
                slab_v.at[pl.ds(first_src, REP * hidden)],
                out_hbm.at[pl.ds(base * hidden, REP * hidden)],
                sem_out)
        ]
        for i in range(1, n_out):
            out_copies.append(pltpu.async_copy(
                slab_v.at[pl.ds(hidden, REP * hidden)],
                out_hbm.at[pl.ds((base + i * REP) * hidden, REP * hidden)],
                sem_out))

        cp_mask.wait()
        for g in range(wpw // LANES):
            acc = mask_v[pl.ds(g * LANES, LANES)]
            for k in range(1, RATIO):
                acc = jnp.minimum(
                    acc, mask_v[pl.ds(k * wpw + g * LANES, LANES)])
            gout_v[pl.ds(g * LANES, LANES)] = acc
        pltpu.sync_copy(gout_v, gmask_hbm.at[pl.ds(base, wpw)])

        for cp in out_copies:
            cp.wait()

    return sc_kernel


def kernel(token_ids, padding_mask, embeds):
    batch, seq_len = token_ids.shape
    hidden = embeds.shape[1]
    num_global = seq_len // RATIO
    rows = batch * seq_len // RATIO
    wpw = rows // NUM_WORKERS
    mask_t = (
        padding_mask.astype(jnp.int32)
        .reshape(NUM_WORKERS, wpw, RATIO)
        .transpose(0, 2, 1)
        .reshape(batch * seq_len)
    )
    out_flat, gmask = _build_sc_kernel(batch, seq_len, hidden)(mask_t, embeds)
    out = out_flat.reshape(batch, num_global, hidden)
    gmask = gmask.reshape(batch, num_global).astype(jnp.bool_)
    return out, gmask


# speedup vs baseline: 1.2666x; 1.2430x over previous
import functools

import jax
import jax.numpy as jnp
from jax import lax
from jax.experimental import pallas as pl
from jax.experimental.pallas import tpu as pltpu
from jax.experimental.pallas import tpu_sc as plsc

RATIO = 16
LANES = 16
NUM_WORKERS = 32
BLK_ROWS = 128


def _build_sc_mask_kernel(rows):
    wpw = rows // NUM_WORKERS
    mesh = plsc.VectorSubcoreMesh(core_axis_name="c", subcore_axis_name="s")

    @functools.partial(
        pl.kernel,
        mesh=mesh,
        out_type=[jax.ShapeDtypeStruct((rows,), jnp.int32)],
        scratch_types=[
            pltpu.VMEM((RATIO * wpw,), jnp.int32),
            pltpu.VMEM((wpw,), jnp.int32),
            pltpu.SemaphoreType.DMA,
        ],
    )
    def sc_mask(mask_hbm, gmask_hbm, mask_v, gout_v, sem_mask):
        wid = lax.axis_index("s") * 2 + lax.axis_index("c")
        base = wid * wpw
        cp = pltpu.async_copy(
            mask_hbm.at[pl.ds(wid * RATIO * wpw, RATIO * wpw)], mask_v,
            sem_mask)
        cp.wait()
        for g in range(wpw // LANES):
            acc = mask_v[pl.ds(g * LANES, LANES)]
            for k in range(1, RATIO):
                acc = jnp.minimum(
                    acc, mask_v[pl.ds(k * wpw + g * LANES, LANES)])
            gout_v[pl.ds(g * LANES, LANES)] = acc
        pltpu.sync_copy(gout_v, gmask_hbm.at[pl.ds(base, wpw)])

    return sc_mask


def _tc_fill(embeds, rows, num_global, hidden):
    def body(emb_ref, out_ref):
        i = pl.program_id(0)
        row = i * BLK_ROWS + lax.broadcasted_iota(
            jnp.int32, (BLK_ROWS, 1), 0)
        is_boundary = (row % num_global) == 0
        out_ref[...] = jnp.where(
            is_boundary, emb_ref[1][None, :], emb_ref[0][None, :])

    return pl.pallas_call(
        body,
        grid=(rows // BLK_ROWS,),
        in_specs=[pl.BlockSpec((2, hidden), lambda i: (0, 0))],
        out_specs=pl.BlockSpec((BLK_ROWS, hidden), lambda i: (i, 0)),
        out_shape=jax.ShapeDtypeStruct((rows, hidden), jnp.float32),
    )(embeds)


def kernel(token_ids, padding_mask, embeds):
    batch, seq_len = token_ids.shape
    hidden = embeds.shape[1]
    num_global = seq_len // RATIO
    rows = batch * num_global
    wpw = rows // NUM_WORKERS
    mask_t = (
        padding_mask.astype(jnp.int32)
        .reshape(NUM_WORKERS, wpw, RATIO)
        .transpose(0, 2, 1)
        .reshape(batch * seq_len)
    )
    (gmask,) = _build_sc_mask_kernel(rows)(mask_t)
    out_flat = _tc_fill(embeds[:2], rows, num_global, hidden)
    out = out_flat.reshape(batch, num_global, hidden)
    gmask = gmask.reshape(batch, num_global).astype(jnp.bool_)
    return out, gmask


# speedup vs baseline: 1.3498x; 1.0657x over previous
import functools

import jax
import jax.numpy as jnp
from jax import lax
from jax.experimental import pallas as pl
from jax.experimental.pallas import tpu as pltpu
from jax.experimental.pallas import tpu_sc as plsc

RATIO = 16
LANES = 16
NUM_WORKERS = 32
BLK_ROWS = 512


def _build_sc_mask_kernel(rows):
    wpw = rows // NUM_WORKERS
    mesh = plsc.VectorSubcoreMesh(core_axis_name="c", subcore_axis_name="s")

    @functools.partial(
        pl.kernel,
        mesh=mesh,
        out_type=[jax.ShapeDtypeStruct((rows,), jnp.int32)],
        scratch_types=[
            pltpu.VMEM((RATIO * wpw,), jnp.int32),
            pltpu.VMEM((wpw,), jnp.int32),
            pltpu.SemaphoreType.DMA,
        ],
    )
    def sc_mask(mask_hbm, gmask_hbm, mask_v, gout_v, sem_mask):
        wid = lax.axis_index("s") * 2 + lax.axis_index("c")
        base = wid * wpw
        cp = pltpu.async_copy(
            mask_hbm.at[pl.ds(wid * RATIO * wpw, RATIO * wpw)], mask_v,
            sem_mask)
        cp.wait()
        for g in range(wpw // LANES):
            acc = mask_v[pl.ds(g * LANES, LANES)]
            for k in range(1, RATIO):
                acc = jnp.minimum(
                    acc, mask_v[pl.ds(k * wpw + g * LANES, LANES)])
            gout_v[pl.ds(g * LANES, LANES)] = acc
        pltpu.sync_copy(gout_v, gmask_hbm.at[pl.ds(base, wpw)])

    return sc_mask


def _tc_fill(embeds, rows, num_global, hidden):
    def body(emb_ref, out_ref):
        i = pl.program_id(0)
        row = i * BLK_ROWS + lax.broadcasted_iota(
            jnp.int32, (BLK_ROWS, 1), 0)
        is_boundary = (row % num_global) == 0
        out_ref[...] = jnp.where(
            is_boundary, emb_ref[1][None, :], emb_ref[0][None, :])

    return pl.pallas_call(
        body,
        grid=(rows // BLK_ROWS,),
        in_specs=[pl.BlockSpec((2, hidden), lambda i: (0, 0))],
        out_specs=pl.BlockSpec((BLK_ROWS, hidden), lambda i: (i, 0)),
        out_shape=jax.ShapeDtypeStruct((rows, hidden), jnp.float32),
    )(embeds)


def kernel(token_ids, padding_mask, embeds):
    batch, seq_len = token_ids.shape
    hidden = embeds.shape[1]
    num_global = seq_len // RATIO
    rows = batch * num_global
    wpw = rows // NUM_WORKERS
    mask_t = (
        padding_mask.astype(jnp.int32)
        .reshape(NUM_WORKERS, wpw, RATIO)
        .transpose(0, 2, 1)
        .reshape(batch * seq_len)
    )
    (gmask,) = _build_sc_mask_kernel(rows)(mask_t)
    out_flat = _tc_fill(embeds[:2], rows, num_global, hidden)
    out = out_flat.reshape(batch, num_global, hidden)
    gmask = gmask.reshape(batch, num_global).astype(jnp.bool_)
    return out, gmask
